# baseline (speedup 1.0000x reference)
import functools

import jax
import jax.numpy as jnp
from jax import lax
from jax.experimental import pallas as pl
from jax.experimental.pallas import tpu as pltpu
from jax.experimental.pallas import tpu_sc as plsc

B = 1024
DM = 128
N2 = 2 * B
ND = 1000
NDP = 1024
NC = 2
NS = 16
NW = NC * NS
L = 16

U_PER = B // NW
I_PER = N2 // NW
CHUNK = 8
NCHUNK = I_PER // CHUNK


def _sc_gather_body(uid_hbm, t_hbm, utab_hbm, itab_hbm, dpad_hbm,
                    uemb_hbm, emb_hbm, dsub_hbm,
                    uidx_v, urows_v, tidx_v, erows_v, r_v, obuf_v, tfull_v,
                    sem_r0, sem_r1, sem_u, sem_e, sem_o0, sem_o1):
    wid = lax.axis_index("c") * NS + lax.axis_index("s")
    ubase = wid * U_PER
    ibase = wid * I_PER
    sem_r = (sem_r0, sem_r1)
    sem_o = (sem_o0, sem_o1)

    pltpu.sync_copy(t_hbm.at[pl.ds(ibase, I_PER)], tidx_v)

    r_descs = [None] * NCHUNK
    o_descs = [None] * NCHUNK
    r_descs[0] = pltpu.async_copy(
        dpad_hbm.at[tidx_v.at[pl.ds(0, CHUNK)]], r_v.at[0], sem_r[0])

    pltpu.sync_copy(t_hbm, tfull_v)

    pltpu.sync_copy(uid_hbm.at[pl.ds(ubase, U_PER)], uidx_v)
    u_dma = pltpu.async_copy(utab_hbm.at[uidx_v], urows_v, sem_u)
    e_dma = pltpu.async_copy(itab_hbm.at[tidx_v], erows_v, sem_e)

    for c in range(NCHUNK):
        pb = c % 2
        if c + 1 < NCHUNK:
            nb = (c + 1) % 2
            r_descs[c + 1] = pltpu.async_copy(
                dpad_hbm.at[tidx_v.at[pl.ds((c + 1) * CHUNK, CHUNK)]],
                r_v.at[nb], sem_r[nb])
        r_descs[c].wait()
        if c >= 2:
            o_descs[c - 2].wait()
        rbuf = r_v.at[pb]
        obuf = obuf_v.at[pb]

        def body(jg, carry, rbuf=rbuf, obuf=obuf):
            cvec = tfull_v[pl.ds(jg * L, L)]
            jc = jg >> 3
            col_in = (jg & 7) * L
            for r in range(CHUNK):
                rvec = jnp.full((L,), r, dtype=jnp.int32)
                obuf[r, jc, pl.ds(col_in, L)] = plsc.load_gather(
                    rbuf, [rvec, cvec])
            return carry

        lax.fori_loop(0, N2 // L, body, None, unroll=2)
        o_descs[c] = pltpu.async_copy(
            obuf_v.at[pb], dsub_hbm.at[pl.ds(ibase + c * CHUNK, CHUNK)],
            sem_o[pb])

    u_dma.wait()
    pltpu.sync_copy(urows_v, uemb_hbm.at[pl.ds(ubase, U_PER)])
    e_dma.wait()
    pltpu.sync_copy(erows_v, emb_hbm.at[pl.ds(ibase, I_PER)])
    o_descs[NCHUNK - 2].wait()
    o_descs[NCHUNK - 1].wait()


def _sc_gather(uid, t, utab, itab, dpad):
    f32 = jnp.float32
    kern = pl.kernel(
        _sc_gather_body,
        out_type=[
            jax.ShapeDtypeStruct((B, DM), f32),
            jax.ShapeDtypeStruct((N2, DM), f32),
            jax.ShapeDtypeStruct((N2, N2 // DM, DM), f32),
        ],
        mesh=plsc.VectorSubcoreMesh(
            core_axis_name="c", subcore_axis_name="s",
            num_cores=NC, num_subcores=NS),
        scratch_types=[
            pltpu.VMEM((U_PER,), jnp.int32),
            pltpu.VMEM((U_PER, DM), f32),
            pltpu.VMEM((I_PER,), jnp.int32),
            pltpu.VMEM((I_PER, DM), f32),
            pltpu.VMEM((2, CHUNK, NDP), f32),
            pltpu.VMEM((2, CHUNK, N2 // DM, DM), f32),
            pltpu.VMEM((N2,), jnp.int32),
            pltpu.SemaphoreType.DMA,
            pltpu.SemaphoreType.DMA,
            pltpu.SemaphoreType.DMA,
            pltpu.SemaphoreType.DMA,
            pltpu.SemaphoreType.DMA,
            pltpu.SemaphoreType.DMA,
        ],
        compiler_params=pltpu.CompilerParams(use_tc_tiling_on_sc=False,
                                             needs_layout_passes=False),
    )
    return kern(uid, t, utab, itab, dpad)


TBLK = 256
NBLK = N2 // TBLK


def _tc_main_body(uemb_ref, emb_ref, dsub_ref, bpp_ref, reg_ref):
    i = pl.program_id(0)

    @pl.when(i == 0)
    def _():
        u = uemb_ref[...]
        diff = emb_ref[:B, :] - emb_ref[B:, :]
        x = jnp.sum(u * diff, axis=1)
        loss = jnp.maximum(-x, 0.0) + jnp.log1p(jnp.exp(-jnp.abs(x)))
        bpp_ref[0, 0] = jnp.sum(loss)
        reg_ref[0, 0] = 0.0

    e = emb_ref[...]
    eb = emb_ref[pl.ds(i * TBLK, TBLK), :]
    gram = lax.dot_general(eb, e, (((1,), (1,)), ((), ())),
                           preferred_element_type=jnp.float32)
    sq_all = jnp.sum(e * e, axis=1)
    sq_b = jnp.sum(eb * eb, axis=1)
    d2 = jnp.maximum(sq_b[:, None] + sq_all[None, :] - 2.0 * gram, 1e-12)
    rows = i * TBLK + lax.broadcasted_iota(jnp.int32, (TBLK, N2), 0)
    cols = lax.broadcasted_iota(jnp.int32, (TBLK, N2), 1)
    dist = jnp.where(cols > rows, jnp.sqrt(d2), 0.0)
    acc = jnp.float32(0.0)
    for jc in range(N2 // DM):
        acc += jnp.sum(dsub_ref[:, jc, :]
                       * lax.slice(dist, (0, jc * DM), (TBLK, (jc + 1) * DM)))
    reg_ref[0, 0] += acc


def _tc_main(uemb, emb, dsub):
    f32 = jnp.float32
    bpp, reg = pl.pallas_call(
        _tc_main_body,
        grid=(NBLK,),
        in_specs=[
            pl.BlockSpec((B, DM), lambda i: (0, 0)),
            pl.BlockSpec((N2, DM), lambda i: (0, 0)),
            pl.BlockSpec((TBLK, N2 // DM, DM), lambda i: (i, 0, 0)),
        ],
        out_specs=[
            pl.BlockSpec(memory_space=pltpu.SMEM),
            pl.BlockSpec(memory_space=pltpu.SMEM),
        ],
        out_shape=[
            jax.ShapeDtypeStruct((1, 1), f32),
            jax.ShapeDtypeStruct((1, 1), f32),
        ],
        compiler_params=pltpu.CompilerParams(
            dimension_semantics=("arbitrary",)),
    )(uemb, emb, dsub)
    return bpp, reg


def kernel(user_id, pos_id, neg_id, user_embedding_weights,
           item_embedding_weights, cosine_distances_D):
    uid = user_id.astype(jnp.int32)
    t = jnp.concatenate([pos_id, neg_id]).astype(jnp.int32)
    dpad = jnp.pad(cosine_distances_D, ((0, 0), (0, NDP - ND)))
    uemb, emb, dsub = _sc_gather(uid, t, user_embedding_weights,
                                 item_embedding_weights, dpad)
    bpp, reg = _tc_main(uemb, emb, dsub)
    return (bpp[0, 0], reg[0, 0])


# speedup vs baseline: 1.0669x; 1.0669x over previous
import functools

import jax
import jax.numpy as jnp
from jax import lax
from jax.experimental import pallas as pl
from jax.experimental.pallas import tpu as pltpu
from jax.experimental.pallas import tpu_sc as plsc

B = 1024
DM = 128
N2 = 2 * B
ND = 1000
NDP = 1024
NC = 2
NS = 16
NW = NC * NS
L = 16

U_PER = B // NW
I_PER = N2 // NW
CHUNK = 8
NCHUNK = I_PER // CHUNK


def _sc_gather_body(uid_hbm, t_hbm, utab_hbm, itab_hbm, dpad_hbm,
                    uemb_hbm, emb_hbm, dsub_hbm,
                    uidx_v, urows_v, tidx_v, erows_v, r_v, obuf_v, tfull_v,
                    sem_r0, sem_r1, sem_u, sem_e, sem_o0, sem_o1):
    wid = lax.axis_index("c") * NS + lax.axis_index("s")
    ubase = wid * U_PER
    ibase = wid * I_PER
    sem_r = (sem_r0, sem_r1)
    sem_o = (sem_o0, sem_o1)

    pltpu.sync_copy(t_hbm.at[pl.ds(ibase, I_PER)], tidx_v)

    r_descs = [None] * NCHUNK
    o_descs = [None] * NCHUNK
    r_descs[0] = pltpu.async_copy(
        dpad_hbm.at[tidx_v.at[pl.ds(0, CHUNK)]], r_v.at[0], sem_r[0])

    pltpu.sync_copy(t_hbm, tfull_v)

    pltpu.sync_copy(uid_hbm.at[pl.ds(ubase, U_PER)], uidx_v)
    u_dma = pltpu.async_copy(utab_hbm.at[uidx_v], urows_v, sem_u)
    e_dma = pltpu.async_copy(itab_hbm.at[tidx_v], erows_v, sem_e)

    for c in range(NCHUNK):
        pb = c % 2
        if c + 1 < NCHUNK:
            nb = (c + 1) % 2
            r_descs[c + 1] = pltpu.async_copy(
                dpad_hbm.at[tidx_v.at[pl.ds((c + 1) * CHUNK, CHUNK)]],
                r_v.at[nb], sem_r[nb])
        r_descs[c].wait()
        if c >= 2:
            o_descs[c - 2].wait()
        rbuf = r_v.at[pb]
        obuf = obuf_v.at[pb]

        def body(jg, carry, rbuf=rbuf, obuf=obuf):
            cvec = tfull_v[pl.ds(jg * L, L)]
            jc = jg >> 3
            col_in = (jg & 7) * L
            for r in range(CHUNK):
                rvec = jnp.full((L,), r, dtype=jnp.int32)
                obuf[r, jc, pl.ds(col_in, L)] = plsc.load_gather(
                    rbuf, [rvec, cvec])
            return carry

        lax.fori_loop(0, N2 // L, body, None, unroll=2)
        o_descs[c] = pltpu.async_copy(
            obuf_v.at[pb], dsub_hbm.at[pl.ds(ibase + c * CHUNK, CHUNK)],
            sem_o[pb])

    u_dma.wait()
    pltpu.sync_copy(urows_v, uemb_hbm.at[pl.ds(ubase, U_PER)])
    e_dma.wait()
    pltpu.sync_copy(erows_v, emb_hbm.at[pl.ds(ibase, I_PER)])
    o_descs[NCHUNK - 2].wait()
    o_descs[NCHUNK - 1].wait()


def _sc_gather(uid, t, utab, itab, dpad):
    f32 = jnp.float32
    kern = pl.kernel(
        _sc_gather_body,
        out_type=[
            jax.ShapeDtypeStruct((B, DM), f32),
            jax.ShapeDtypeStruct((N2, DM), f32),
            jax.ShapeDtypeStruct((N2, N2 // DM, DM), f32),
        ],
        mesh=plsc.VectorSubcoreMesh(
            core_axis_name="c", subcore_axis_name="s",
            num_cores=NC, num_subcores=NS),
        scratch_types=[
            pltpu.VMEM((U_PER,), jnp.int32),
            pltpu.VMEM((U_PER, DM), f32),
            pltpu.VMEM((I_PER,), jnp.int32),
            pltpu.VMEM((I_PER, DM), f32),
            pltpu.VMEM((2, CHUNK, NDP), f32),
            pltpu.VMEM((2, CHUNK, N2 // DM, DM), f32),
            pltpu.VMEM((N2,), jnp.int32),
            pltpu.SemaphoreType.DMA,
            pltpu.SemaphoreType.DMA,
            pltpu.SemaphoreType.DMA,
            pltpu.SemaphoreType.DMA,
            pltpu.SemaphoreType.DMA,
            pltpu.SemaphoreType.DMA,
        ],
        compiler_params=pltpu.CompilerParams(use_tc_tiling_on_sc=False,
                                             needs_layout_passes=False),
    )
    return kern(uid, t, utab, itab, dpad)


TBLK = 256
NBLK = N2 // TBLK


def _tc_main_body(uemb_ref, emb_ref, dsub_ref, bpp_ref, reg_ref):
    i = pl.program_id(0)

    @pl.when(i == 0)
    def _():
        u = uemb_ref[...]
        diff = emb_ref[:B, :] - emb_ref[B:, :]
        x = jnp.sum(u * diff, axis=1)
        loss = jnp.maximum(-x, 0.0) + jnp.log1p(jnp.exp(-jnp.abs(x)))
        bpp_ref[0, 0] = jnp.sum(loss)
        reg_ref[0, 0] = 0.0

    e = emb_ref[...]
    eb = emb_ref[pl.ds(i * TBLK, TBLK), :]
    gram = lax.dot_general(eb, e, (((1,), (1,)), ((), ())),
                           preferred_element_type=jnp.float32)
    sq_all = jnp.sum(e * e, axis=1)
    sq_b = jnp.sum(eb * eb, axis=1)
    d2 = jnp.maximum(sq_b[:, None] + sq_all[None, :] - 2.0 * gram, 1e-12)
    rows = i * TBLK + lax.broadcasted_iota(jnp.int32, (TBLK, N2), 0)
    cols = lax.broadcasted_iota(jnp.int32, (TBLK, N2), 1)
    dist = jnp.where(cols > rows, jnp.sqrt(d2), 0.0)
    acc = dsub_ref[:, 0, :] * lax.slice(dist, (0, 0), (TBLK, DM))
    for jc in range(1, N2 // DM):
        acc += (dsub_ref[:, jc, :]
                * lax.slice(dist, (0, jc * DM), (TBLK, (jc + 1) * DM)))
    reg_ref[0, 0] += jnp.sum(acc)


def _tc_main(uemb, emb, dsub):
    f32 = jnp.float32
    bpp, reg = pl.pallas_call(
        _tc_main_body,
        grid=(NBLK,),
        in_specs=[
            pl.BlockSpec((B, DM), lambda i: (0, 0)),
            pl.BlockSpec((N2, DM), lambda i: (0, 0)),
            pl.BlockSpec((TBLK, N2 // DM, DM), lambda i: (i, 0, 0)),
        ],
        out_specs=[
            pl.BlockSpec(memory_space=pltpu.SMEM),
            pl.BlockSpec(memory_space=pltpu.SMEM),
        ],
        out_shape=[
            jax.ShapeDtypeStruct((1, 1), f32),
            jax.ShapeDtypeStruct((1, 1), f32),
        ],
        compiler_params=pltpu.CompilerParams(
            dimension_semantics=("arbitrary",)),
    )(uemb, emb, dsub)
    return bpp, reg


def kernel(user_id, pos_id, neg_id, user_embedding_weights,
           item_embedding_weights, cosine_distances_D):
    uid = user_id.astype(jnp.int32)
    t = jnp.concatenate([pos_id, neg_id]).astype(jnp.int32)
    dpad = jnp.pad(cosine_distances_D, ((0, 0), (0, NDP - ND)))
    uemb, emb, dsub = _sc_gather(uid, t, user_embedding_weights,
                                 item_embedding_weights, dpad)
    bpp, reg = _tc_main(uemb, emb, dsub)
    return (bpp[0, 0], reg[0, 0])


# speedup vs baseline: 1.5072x; 1.4127x over previous
import functools

import jax
import jax.numpy as jnp
from jax import lax
from jax.experimental import pallas as pl
from jax.experimental.pallas import tpu as pltpu
from jax.experimental.pallas import tpu_sc as plsc

B = 1024
DM = 128
N2 = 2 * B
ND = 1000
NDP = 1024
NC = 2
NS = 16
NW = NC * NS
L = 16

U_PER = B // NW
I_PER = N2 // NW
CHUNK = 8
NCHUNK = I_PER // CHUNK


def _sc_gather_body(uid_hbm, t_hbm, utab_hbm, itab_hbm, dpad_hbm,
                    uemb_hbm, emb_hbm, dsub_hbm,
                    uidx_v, urows_v, tidx_v, erows_v, r_v, obuf_v, tfull_v,
                    sem_r0, sem_r1, sem_u, sem_e, sem_o0, sem_o1):
    wid = lax.axis_index("c") * NS + lax.axis_index("s")
    ubase = wid * U_PER
    ibase = wid * I_PER
    sem_r = (sem_r0, sem_r1)
    sem_o = (sem_o0, sem_o1)

    pltpu.sync_copy(t_hbm.at[pl.ds(ibase, I_PER)], tidx_v)

    r_descs = [None] * NCHUNK
    o_descs = [None] * NCHUNK
    r_descs[0] = pltpu.async_copy(
        dpad_hbm.at[tidx_v.at[pl.ds(0, CHUNK)]], r_v.at[0], sem_r[0])

    pltpu.sync_copy(t_hbm, tfull_v)

    pltpu.sync_copy(uid_hbm.at[pl.ds(ubase, U_PER)], uidx_v)
    u_dma = pltpu.async_copy(utab_hbm.at[uidx_v], urows_v, sem_u)
    e_dma = pltpu.async_copy(itab_hbm.at[tidx_v], erows_v, sem_e)

    for c in range(NCHUNK):
        pb = c % 2
        if c + 1 < NCHUNK:
            nb = (c + 1) % 2
            r_descs[c + 1] = pltpu.async_copy(
                dpad_hbm.at[tidx_v.at[pl.ds((c + 1) * CHUNK, CHUNK)]],
                r_v.at[nb], sem_r[nb])
        r_descs[c].wait()
        if c >= 2:
            o_descs[c - 2].wait()
        rbuf = r_v.at[pb]
        obuf = obuf_v.at[pb]

        @plsc.parallel_loop(0, N2 // L, step=1, unroll=2)
        def _(jg, rbuf=rbuf, obuf=obuf):
            cvec = tfull_v[pl.ds(jg * L, L)]
            jc = jg >> 3
            col_in = (jg & 7) * L
            for r in range(CHUNK):
                rvec = jnp.full((L,), r, dtype=jnp.int32)
                obuf[r, jc, pl.ds(col_in, L)] = plsc.load_gather(
                    rbuf, [rvec, cvec])
        o_descs[c] = pltpu.async_copy(
            obuf_v.at[pb], dsub_hbm.at[pl.ds(ibase + c * CHUNK, CHUNK)],
            sem_o[pb])

    u_dma.wait()
    pltpu.sync_copy(urows_v, uemb_hbm.at[pl.ds(ubase, U_PER)])
    e_dma.wait()
    pltpu.sync_copy(erows_v, emb_hbm.at[pl.ds(ibase, I_PER)])
    o_descs[NCHUNK - 2].wait()
    o_descs[NCHUNK - 1].wait()


def _sc_gather(uid, t, utab, itab, dpad):
    f32 = jnp.float32
    kern = pl.kernel(
        _sc_gather_body,
        out_type=[
            jax.ShapeDtypeStruct((B, DM), f32),
            jax.ShapeDtypeStruct((N2, DM), f32),
            jax.ShapeDtypeStruct((N2, N2 // DM, DM), f32),
        ],
        mesh=plsc.VectorSubcoreMesh(
            core_axis_name="c", subcore_axis_name="s",
            num_cores=NC, num_subcores=NS),
        scratch_types=[
            pltpu.VMEM((U_PER,), jnp.int32),
            pltpu.VMEM((U_PER, DM), f32),
            pltpu.VMEM((I_PER,), jnp.int32),
            pltpu.VMEM((I_PER, DM), f32),
            pltpu.VMEM((2, CHUNK, NDP), f32),
            pltpu.VMEM((2, CHUNK, N2 // DM, DM), f32),
            pltpu.VMEM((N2,), jnp.int32),
            pltpu.SemaphoreType.DMA,
            pltpu.SemaphoreType.DMA,
            pltpu.SemaphoreType.DMA,
            pltpu.SemaphoreType.DMA,
            pltpu.SemaphoreType.DMA,
            pltpu.SemaphoreType.DMA,
        ],
        compiler_params=pltpu.CompilerParams(use_tc_tiling_on_sc=False,
                                             needs_layout_passes=False),
    )
    return kern(uid, t, utab, itab, dpad)


TBLK = 256
NBLK = N2 // TBLK


def _tc_main_body(uemb_ref, emb_ref, dsub_ref, bpp_ref, reg_ref):
    i = pl.program_id(0)

    @pl.when(i == 0)
    def _():
        u = uemb_ref[...]
        diff = emb_ref[:B, :] - emb_ref[B:, :]
        x = jnp.sum(u * diff, axis=1)
        loss = jnp.maximum(-x, 0.0) + jnp.log1p(jnp.exp(-jnp.abs(x)))
        bpp_ref[0, 0] = jnp.sum(loss)
        reg_ref[0, 0] = 0.0

    e = emb_ref[...]
    eb = emb_ref[pl.ds(i * TBLK, TBLK), :]
    gram = lax.dot_general(eb, e, (((1,), (1,)), ((), ())),
                           preferred_element_type=jnp.float32)
    sq_all = jnp.sum(e * e, axis=1)
    sq_b = jnp.sum(eb * eb, axis=1)
    d2 = jnp.maximum(sq_b[:, None] + sq_all[None, :] - 2.0 * gram, 1e-12)
    rows = i * TBLK + lax.broadcasted_iota(jnp.int32, (TBLK, N2), 0)
    cols = lax.broadcasted_iota(jnp.int32, (TBLK, N2), 1)
    dist = jnp.where(cols > rows, jnp.sqrt(d2), 0.0)
    acc = dsub_ref[:, 0, :] * lax.slice(dist, (0, 0), (TBLK, DM))
    for jc in range(1, N2 // DM):
        acc += (dsub_ref[:, jc, :]
                * lax.slice(dist, (0, jc * DM), (TBLK, (jc + 1) * DM)))
    reg_ref[0, 0] += jnp.sum(acc)


def _tc_main(uemb, emb, dsub):
    f32 = jnp.float32
    bpp, reg = pl.pallas_call(
        _tc_main_body,
        grid=(NBLK,),
        in_specs=[
            pl.BlockSpec((B, DM), lambda i: (0, 0)),
            pl.BlockSpec((N2, DM), lambda i: (0, 0)),
            pl.BlockSpec((TBLK, N2 // DM, DM), lambda i: (i, 0, 0)),
        ],
        out_specs=[
            pl.BlockSpec(memory_space=pltpu.SMEM),
            pl.BlockSpec(memory_space=pltpu.SMEM),
        ],
        out_shape=[
            jax.ShapeDtypeStruct((1, 1), f32),
            jax.ShapeDtypeStruct((1, 1), f32),
        ],
        compiler_params=pltpu.CompilerParams(
            dimension_semantics=("arbitrary",)),
    )(uemb, emb, dsub)
    return bpp, reg


def kernel(user_id, pos_id, neg_id, user_embedding_weights,
           item_embedding_weights, cosine_distances_D):
    uid = user_id.astype(jnp.int32)
    t = jnp.concatenate([pos_id, neg_id]).astype(jnp.int32)
    dpad = jnp.pad(cosine_distances_D, ((0, 0), (0, NDP - ND)))
    uemb, emb, dsub = _sc_gather(uid, t, user_embedding_weights,
                                 item_embedding_weights, dpad)
    bpp, reg = _tc_main(uemb, emb, dsub)
    return (bpp[0, 0], reg[0, 0])
